# baseline (speedup 1.0000x reference)
import functools
import math

import jax
import jax.numpy as jnp
from jax.experimental import pallas as pl
from jax.experimental.pallas import tpu as pltpu

_B, _N, _M, _K = 4, 8192, 2048, 64
_INFEA = 16
_DIST2 = 0.4 * 0.4
_EPS = 1e-5
_C = 32
_MB = 32
_PB = _MB * _K
_P_TOT = _B * _M * _K


def _x_tile(g, qpad):
    qb = jnp.broadcast_to(qpad[:, None, :], (_MB, _K, _C)).reshape(_PB, _C)
    return g - qb


def _stats1_kernel(g_ref, npc_ref, acc_ref):
    b = pl.program_id(0)
    mi = pl.program_id(1)

    @pl.when(jnp.logical_and(b == 0, mi == 0))
    def _():
        acc_ref[...] = jnp.zeros_like(acc_ref)

    x = _x_tile(g_ref[...], npc_ref[0])
    gram = jax.lax.dot_general(x, x, (((0,), (0,)), ((), ())),
                               preferred_element_type=jnp.float32)
    s1 = jnp.sum(x, axis=0)
    acc_ref[0:_C, :] += gram
    acc_ref[_C:_C + 1, :] += s1[None, :]


def _stats2_kernel(g_ref, npc_ref, a1_ref, c1_ref, acc_ref):
    b = pl.program_id(0)
    mi = pl.program_id(1)

    @pl.when(jnp.logical_and(b == 0, mi == 0))
    def _():
        acc_ref[...] = jnp.zeros_like(acc_ref)

    x = _x_tile(g_ref[...], npc_ref[0])
    h1 = jnp.maximum(
        jax.lax.dot_general(x, a1_ref[...], (((1,), (1,)), ((), ())),
                            preferred_element_type=jnp.float32)
        + c1_ref[...], 0.0)
    gram = jax.lax.dot_general(h1, h1, (((0,), (0,)), ((), ())),
                               preferred_element_type=jnp.float32)
    acc_ref[0:_C, :] += gram
    acc_ref[_C:_C + 1, :] += jnp.sum(h1, axis=0)[None, :]


def _stats3_kernel(g_ref, npc_ref, a1_ref, c1_ref, a2_ref, c2_ref, acc_ref):
    b = pl.program_id(0)
    mi = pl.program_id(1)

    @pl.when(jnp.logical_and(b == 0, mi == 0))
    def _():
        acc_ref[...] = jnp.zeros_like(acc_ref)

    x = _x_tile(g_ref[...], npc_ref[0])
    h1 = jnp.maximum(
        jax.lax.dot_general(x, a1_ref[...], (((1,), (1,)), ((), ())),
                            preferred_element_type=jnp.float32)
        + c1_ref[...], 0.0)
    h2 = jnp.maximum(
        jax.lax.dot_general(h1, a2_ref[...], (((1,), (1,)), ((), ())),
                            preferred_element_type=jnp.float32)
        + c2_ref[...], 0.0)
    gram = jax.lax.dot_general(h2, h2, (((0,), (0,)), ((), ())),
                               preferred_element_type=jnp.float32)
    acc_ref[0:_C, :] += gram
    acc_ref[_C:_C + 1, :] += jnp.sum(h2, axis=0)[None, :]


def _final_kernel(g_ref, npc_ref, a1_ref, c1_ref, a2_ref, c2_ref,
                  a3_ref, c3_ref, valid_ref, out_ref):
    b = pl.program_id(0)
    x = _x_tile(g_ref[...], npc_ref[0])
    h1 = jnp.maximum(
        jax.lax.dot_general(x, a1_ref[...], (((1,), (1,)), ((), ())),
                            preferred_element_type=jnp.float32)
        + c1_ref[...], 0.0)
    h2 = jnp.maximum(
        jax.lax.dot_general(h1, a2_ref[...], (((1,), (1,)), ((), ())),
                            preferred_element_type=jnp.float32)
        + c2_ref[...], 0.0)
    y = jnp.maximum(
        jax.lax.dot_general(h2, a3_ref[...], (((1,), (1,)), ((), ())),
                            preferred_element_type=jnp.float32)
        + c3_ref[...], 0.0)
    vrow = valid_ref[0, 0]
    vmask = jnp.broadcast_to(vrow.reshape(_MB, 1, 1), (_MB, _K, 1))
    y = y * vmask.reshape(_PB, 1)
    out_ref[0] = y.T.reshape(64, _MB, _K)


def _fold(acc, W, bvec, gvec, beta, cin):
    n = float(_P_TOT)
    gram = acc[0:_C, 0:_C] / n
    mu = acc[_C, 0:_C] / n
    Wp = jnp.zeros((W.shape[0], _C), jnp.float32).at[:, :cin].set(W)
    wmu = Wp @ mu
    mean_y = wmu + bvec
    e_yy = jnp.einsum('oc,cd,od->o', Wp, gram, Wp) + 2.0 * bvec * wmu + bvec * bvec
    var_y = e_yy - mean_y * mean_y
    a = gvec * jax.lax.rsqrt(var_y + _EPS)
    A = a[:, None] * Wp
    c = a * bvec + beta - a * mean_y
    return A, c[None, :]


def _mlp_passes(G, npc32, valid, W1, b1, g1, beta1, W2, b2, g2, beta2,
                W3, b3, g3, beta3):
    valid = valid.reshape(_B, _M // _MB, 1, _MB)
    grid = (_B, _M // _MB)
    g_spec = pl.BlockSpec((_PB, _C), lambda b, mi: (b * (_M // _MB) + mi, 0))
    npc_spec = pl.BlockSpec((1, _MB, _C), lambda b, mi: (b, mi, 0))
    acc_shape = jax.ShapeDtypeStruct((_C + 8, _C), jnp.float32)
    acc_spec = pl.BlockSpec((_C + 8, _C), lambda b, mi: (0, 0))
    mat_spec = pl.BlockSpec((_C, _C), lambda b, mi: (0, 0))
    c_spec = pl.BlockSpec((1, _C), lambda b, mi: (0, 0))

    acc1 = pl.pallas_call(
        _stats1_kernel, grid=grid,
        in_specs=[g_spec, npc_spec],
        out_specs=acc_spec, out_shape=acc_shape,
    )(G, npc32)
    A1, c1 = _fold(acc1, W1, b1, g1, beta1, 3 + _INFEA)

    acc2 = pl.pallas_call(
        _stats2_kernel, grid=grid,
        in_specs=[g_spec, npc_spec, mat_spec, c_spec],
        out_specs=acc_spec, out_shape=acc_shape,
    )(G, npc32, A1, c1)
    A2, c2 = _fold(acc2, W2, b2, g2, beta2, 32)

    acc3 = pl.pallas_call(
        _stats3_kernel, grid=grid,
        in_specs=[g_spec, npc_spec, mat_spec, c_spec, mat_spec, c_spec],
        out_specs=acc_shape and acc_spec, out_shape=acc_shape,
    )(G, npc32, A1, c1, A2, c2)
    A3, c3 = _fold(acc3, W3, b3, g3, beta3, 32)
    A3p = jnp.zeros((64, _C), jnp.float32).at[:, :].set(A3)

    out = pl.pallas_call(
        _final_kernel, grid=grid,
        in_specs=[g_spec, npc_spec, mat_spec, c_spec, mat_spec, c_spec,
                  pl.BlockSpec((64, _C), lambda b, mi: (0, 0)),
                  pl.BlockSpec((1, 64), lambda b, mi: (0, 0)),
                  pl.BlockSpec((1, 1, 1, _MB), lambda b, mi: (b, mi, 0, 0))],
        out_specs=pl.BlockSpec((1, 64, _MB, _K), lambda b, mi: (b, 0, mi, 0)),
        out_shape=jax.ShapeDtypeStruct((_B, 64, _M, _K), jnp.float32),
    )(G, npc32, A1, c1, A2, c2, A3p, c3, valid)
    return out


def kernel(pc, feat, new_pc, W1, b1, g1, beta1, W2, b2, g2, beta2,
           W3, b3, g3, beta3):
    d2 = (jnp.sum(new_pc * new_pc, axis=1)[:, :, None]
          + jnp.sum(pc * pc, axis=1)[:, None, :]
          - 2.0 * jnp.einsum('bcm,bcn->bmn', new_pc, pc))
    mask = d2 < _DIST2
    counts = jnp.sum(mask.astype(jnp.int32), axis=-1)
    scores = jnp.where(mask, jnp.arange(_N, dtype=jnp.int32)[None, None, :], _N)
    _, idx = jax.lax.top_k(-scores, _K)
    pos = jnp.arange(_K, dtype=jnp.int32)
    valid_slot = pos[None, None, :] < counts[:, :, None]
    idx = jnp.where(valid_slot, idx, 0)
    valid = (counts > 0).astype(jnp.float32)

    idx_flat = idx.reshape(_B, 1, _M * _K)
    gpc = jnp.take_along_axis(pc, jnp.broadcast_to(idx_flat, (_B, 3, _M * _K)),
                              axis=2)
    gfeat = jnp.take_along_axis(
        feat, jnp.broadcast_to(idx_flat, (_B, _INFEA, _M * _K)), axis=2)
    rows = jnp.concatenate([gpc, gfeat], axis=1)
    rows = jnp.moveaxis(rows, 1, 2).reshape(_P_TOT, 3 + _INFEA)
    G = jnp.zeros((_P_TOT, _C), jnp.float32).at[:, :3 + _INFEA].set(rows)

    npc32 = jnp.zeros((_B, _M, _C), jnp.float32).at[:, :, :3].set(
        jnp.moveaxis(new_pc, 1, 2))

    return _mlp_passes(G, npc32, valid, W1, b1, g1, beta1,
                       W2, b2, g2, beta2, W3, b3, g3, beta3)


# speedup vs baseline: 35.7597x; 35.7597x over previous
import functools
import math

import jax
import jax.numpy as jnp
from jax import lax
from jax.experimental import pallas as pl
from jax.experimental.pallas import tpu as pltpu
from jax.experimental.pallas import tpu_sc as plsc

_B, _N, _M, _K = 4, 8192, 2048, 64
_INFEA = 16
_DIST2 = 0.4 * 0.4
_EPS = 1e-5
_C = 32
_MB = 32
_PB = _MB * _K
_P_TOT = _B * _M * _K


def _x_tile(g, qpad):
    qb = jnp.broadcast_to(qpad[:, None, :], (_MB, _K, _C)).reshape(_PB, _C)
    return g - qb


def _stats1_kernel(g_ref, npc_ref, acc_ref):
    b = pl.program_id(0)
    mi = pl.program_id(1)

    @pl.when(jnp.logical_and(b == 0, mi == 0))
    def _():
        acc_ref[...] = jnp.zeros_like(acc_ref)

    x = _x_tile(g_ref[...], npc_ref[0])
    gram = jax.lax.dot_general(x, x, (((0,), (0,)), ((), ())),
                               preferred_element_type=jnp.float32)
    s1 = jnp.sum(x, axis=0)
    acc_ref[0:_C, :] += gram
    acc_ref[_C:_C + 1, :] += s1[None, :]


def _stats2_kernel(g_ref, npc_ref, a1_ref, c1_ref, acc_ref):
    b = pl.program_id(0)
    mi = pl.program_id(1)

    @pl.when(jnp.logical_and(b == 0, mi == 0))
    def _():
        acc_ref[...] = jnp.zeros_like(acc_ref)

    x = _x_tile(g_ref[...], npc_ref[0])
    h1 = jnp.maximum(
        jax.lax.dot_general(x, a1_ref[...], (((1,), (1,)), ((), ())),
                            preferred_element_type=jnp.float32)
        + c1_ref[...], 0.0)
    gram = jax.lax.dot_general(h1, h1, (((0,), (0,)), ((), ())),
                               preferred_element_type=jnp.float32)
    acc_ref[0:_C, :] += gram
    acc_ref[_C:_C + 1, :] += jnp.sum(h1, axis=0)[None, :]


def _stats3_kernel(g_ref, npc_ref, a1_ref, c1_ref, a2_ref, c2_ref, acc_ref):
    b = pl.program_id(0)
    mi = pl.program_id(1)

    @pl.when(jnp.logical_and(b == 0, mi == 0))
    def _():
        acc_ref[...] = jnp.zeros_like(acc_ref)

    x = _x_tile(g_ref[...], npc_ref[0])
    h1 = jnp.maximum(
        jax.lax.dot_general(x, a1_ref[...], (((1,), (1,)), ((), ())),
                            preferred_element_type=jnp.float32)
        + c1_ref[...], 0.0)
    h2 = jnp.maximum(
        jax.lax.dot_general(h1, a2_ref[...], (((1,), (1,)), ((), ())),
                            preferred_element_type=jnp.float32)
        + c2_ref[...], 0.0)
    gram = jax.lax.dot_general(h2, h2, (((0,), (0,)), ((), ())),
                               preferred_element_type=jnp.float32)
    acc_ref[0:_C, :] += gram
    acc_ref[_C:_C + 1, :] += jnp.sum(h2, axis=0)[None, :]


def _final_kernel(g_ref, npc_ref, a1_ref, c1_ref, a2_ref, c2_ref,
                  a3_ref, c3_ref, valid_ref, out_ref):
    b = pl.program_id(0)
    x = _x_tile(g_ref[...], npc_ref[0])
    h1 = jnp.maximum(
        jax.lax.dot_general(x, a1_ref[...], (((1,), (1,)), ((), ())),
                            preferred_element_type=jnp.float32)
        + c1_ref[...], 0.0)
    h2 = jnp.maximum(
        jax.lax.dot_general(h1, a2_ref[...], (((1,), (1,)), ((), ())),
                            preferred_element_type=jnp.float32)
        + c2_ref[...], 0.0)
    y = jnp.maximum(
        jax.lax.dot_general(h2, a3_ref[...], (((1,), (1,)), ((), ())),
                            preferred_element_type=jnp.float32)
        + c3_ref[...], 0.0)
    vrow = valid_ref[0, 0]
    vmask = jnp.broadcast_to(vrow.reshape(_MB, 1, 1), (_MB, _K, 1))
    y = y * vmask.reshape(_PB, 1)
    out_ref[0] = y.T.reshape(64, _MB, _K)


def _fold(acc, W, bvec, gvec, beta, cin):
    n = float(_P_TOT)
    gram = acc[0:_C, 0:_C] / n
    mu = acc[_C, 0:_C] / n
    Wp = jnp.zeros((W.shape[0], _C), jnp.float32).at[:, :cin].set(W)
    wmu = Wp @ mu
    mean_y = wmu + bvec
    e_yy = jnp.einsum('oc,cd,od->o', Wp, gram, Wp) + 2.0 * bvec * wmu + bvec * bvec
    var_y = e_yy - mean_y * mean_y
    a = gvec * jax.lax.rsqrt(var_y + _EPS)
    A = a[:, None] * Wp
    c = a * bvec + beta - a * mean_y
    return A, c[None, :]


def _mlp_passes(G, npc32, valid, W1, b1, g1, beta1, W2, b2, g2, beta2,
                W3, b3, g3, beta3):
    valid = valid.reshape(_B, _M // _MB, 1, _MB)
    grid = (_B, _M // _MB)
    g_spec = pl.BlockSpec((_PB, _C), lambda b, mi: (b * (_M // _MB) + mi, 0))
    npc_spec = pl.BlockSpec((1, _MB, _C), lambda b, mi: (b, mi, 0))
    acc_shape = jax.ShapeDtypeStruct((_C + 8, _C), jnp.float32)
    acc_spec = pl.BlockSpec((_C + 8, _C), lambda b, mi: (0, 0))
    mat_spec = pl.BlockSpec((_C, _C), lambda b, mi: (0, 0))
    c_spec = pl.BlockSpec((1, _C), lambda b, mi: (0, 0))

    acc1 = pl.pallas_call(
        _stats1_kernel, grid=grid,
        in_specs=[g_spec, npc_spec],
        out_specs=acc_spec, out_shape=acc_shape,
    )(G, npc32)
    A1, c1 = _fold(acc1, W1, b1, g1, beta1, 3 + _INFEA)

    acc2 = pl.pallas_call(
        _stats2_kernel, grid=grid,
        in_specs=[g_spec, npc_spec, mat_spec, c_spec],
        out_specs=acc_spec, out_shape=acc_shape,
    )(G, npc32, A1, c1)
    A2, c2 = _fold(acc2, W2, b2, g2, beta2, 32)

    acc3 = pl.pallas_call(
        _stats3_kernel, grid=grid,
        in_specs=[g_spec, npc_spec, mat_spec, c_spec, mat_spec, c_spec],
        out_specs=acc_shape and acc_spec, out_shape=acc_shape,
    )(G, npc32, A1, c1, A2, c2)
    A3, c3 = _fold(acc3, W3, b3, g3, beta3, 32)
    A3p = jnp.zeros((64, _C), jnp.float32).at[:, :].set(A3)

    out = pl.pallas_call(
        _final_kernel, grid=grid,
        in_specs=[g_spec, npc_spec, mat_spec, c_spec, mat_spec, c_spec,
                  pl.BlockSpec((64, _C), lambda b, mi: (0, 0)),
                  pl.BlockSpec((1, 64), lambda b, mi: (0, 0)),
                  pl.BlockSpec((1, 1, 1, _MB), lambda b, mi: (b, mi, 0, 0))],
        out_specs=pl.BlockSpec((1, 64, _MB, _K), lambda b, mi: (b, 0, mi, 0)),
        out_shape=jax.ShapeDtypeStruct((_B, 64, _M, _K), jnp.float32),
    )(G, npc32, A1, c1, A2, c2, A3p, c3, valid)
    return out


_NB = 2048


def _prep_kernel(pc_ref, feat_ref, p_ref):
    cat = jnp.concatenate(
        [pc_ref[0], feat_ref[0],
         jnp.zeros((_C - 3 - _INFEA, _NB), jnp.float32)], axis=0)
    p_ref[...] = cat.T


def _build_point_table(pc, feat):
    grid = (_B, _N // _NB)
    return pl.pallas_call(
        _prep_kernel, grid=grid,
        in_specs=[pl.BlockSpec((1, 3, _NB), lambda b, ni: (b, 0, ni)),
                  pl.BlockSpec((1, _INFEA, _NB), lambda b, ni: (b, 0, ni))],
        out_specs=pl.BlockSpec((_NB, _C), lambda b, ni: (b * (_N // _NB) + ni, 0)),
        out_shape=jax.ShapeDtypeStruct((_B * _N, _C), jnp.float32),
    )(pc, feat)


def _pack_kernel(c_ref, out_ref):
    x, y, z = c_ref[0, 0], c_ref[0, 1], c_ref[0, 2]
    r = c_ref[0].astype(jnp.bfloat16).astype(jnp.float32)
    s = (x * x + y * y) + z * z
    out_ref[0] = jnp.concatenate([r, s[None, :]], axis=0)


def _pack4(arr, L):
    nb = min(L, 2048)
    grid = (_B, L // nb)
    out = pl.pallas_call(
        _pack_kernel, grid=grid,
        in_specs=[pl.BlockSpec((1, 3, nb), lambda b, ni: (b, 0, ni))],
        out_specs=pl.BlockSpec((1, 4, nb), lambda b, ni: (b, 0, ni)),
        out_shape=jax.ShapeDtypeStruct((_B, 4, L), jnp.float32),
    )(arr)
    return out.reshape(_B * 4, L)


_NC, _NS = 2, 16
_NW = _NC * _NS
_CPW = (_B * _M) // _NW
_SEG = _M // (_NW // _B)
_NCHUNK = _N // 16


def _sc_query_gather(pc, new_pc, ptab):
    mesh = plsc.VectorSubcoreMesh(core_axis_name="c", subcore_axis_name="s")

    @functools.partial(
        pl.kernel,
        out_type=(jax.ShapeDtypeStruct((_P_TOT, _C), jnp.float32),
                  jax.ShapeDtypeStruct((_B * _M,), jnp.float32)),
        mesh=mesh,
        compiler_params=pltpu.CompilerParams(needs_layout_passes=False,
                                             use_tc_tiling_on_sc=False),
        scratch_types=[
            pltpu.VMEM((4 * _N,), jnp.float32),
            pltpu.VMEM((4 * _SEG,), jnp.float32),
            pltpu.VMEM((96,), jnp.int32),
            pltpu.VMEM((_K,), jnp.int32),
            pltpu.VMEM((_K, _C), jnp.float32),
            pltpu.VMEM((_SEG,), jnp.float32),
            pltpu.SemaphoreType.DMA,
        ],
    )
    def sck(pc_hbm, npc_hbm, ptab_hbm, g_hbm, valid_hbm,
            pcx, npcs, idxbuf, gidx, rows, flags, sem):
        wid = lax.axis_index("s") * _NC + lax.axis_index("c")
        b = wid // (_NW // _B)
        seg = wid % (_NW // _B)
        m0 = seg * _SEG
        bn = b * _N
        for r in range(4):
            pltpu.sync_copy(pc_hbm.at[b * 4 + r], pcx.at[pl.ds(r * _N, _N)])
            pltpu.sync_copy(npc_hbm.at[b * 4 + r, pl.ds(m0, _SEG)],
                            npcs.at[pl.ds(r * _SEG, _SEG)])
        iota = lax.iota(jnp.int32, 16)
        lane0 = iota == 0
        zeros16 = jnp.zeros((16,), jnp.int32)

        def per_centroid(mi, _):
            mi16 = jnp.full((16,), mi, jnp.int32)
            qx = plsc.load_gather(npcs, [mi16])
            qy = plsc.load_gather(npcs, [mi16 + _SEG])
            qz = plsc.load_gather(npcs, [mi16 + 2 * _SEG])
            sq = plsc.load_gather(npcs, [mi16 + 3 * _SEG])
            for j in range(6):
                idxbuf[pl.ds(j * 16, 16)] = zeros16

            def chunk(nc, tot):
                n0 = nc * 16
                px = pcx[pl.ds(n0, 16)]
                py = pcx[pl.ds(_N + n0, 16)]
                pz = pcx[pl.ds(2 * _N + n0, 16)]
                sp = pcx[pl.ds(3 * _N + n0, 16)]
                dot = px * qx + py * qy + pz * qz
                d2 = sq + sp - 2.0 * dot
                msk = d2 < _DIST2
                m32 = msk.astype(jnp.int32)
                r = plsc.cumsum(m32)
                off = jnp.minimum(tot, _K)
                smask = jnp.logical_and(msk, (r + off) <= _K)
                plsc.store_compressed(idxbuf.at[pl.ds(off, 16)],
                                      iota + n0, mask=smask)
                return tot + jnp.sum(m32)

            total = lax.fori_loop(0, _NCHUNK, chunk, jnp.int32(0))

            for j in range(_K // 16):
                gidx[pl.ds(j * 16, 16)] = idxbuf[pl.ds(j * 16, 16)] + bn
            pltpu.async_copy(ptab_hbm.at[gidx], rows, sem).wait()
            rowbase = (b * _M + m0 + mi) * _K
            pltpu.sync_copy(rows, g_hbm.at[pl.ds(rowbase, _K)])
            flagv = jnp.where(jnp.full((16,), total) > 0, 1.0, 0.0)
            plsc.store_scatter(flags, [jnp.full((16,), mi, jnp.int32)],
                               flagv, mask=lane0)
            return 0

        lax.fori_loop(0, _SEG, per_centroid, 0)
        pltpu.sync_copy(flags, valid_hbm.at[pl.ds(b * _M + m0, _SEG)])

    return sck(_pack4(pc, _N), _pack4(new_pc, _M), ptab)


def kernel(pc, feat, new_pc, W1, b1, g1, beta1, W2, b2, g2, beta2,
           W3, b3, g3, beta3):
    ptab = _build_point_table(pc, feat)
    G, validf = _sc_query_gather(pc, new_pc, ptab)
    valid = validf.reshape(_B, _M)

    npc32 = jnp.zeros((_B, _M, _C), jnp.float32).at[:, :, :3].set(
        jnp.moveaxis(new_pc, 1, 2))

    return _mlp_passes(G, npc32, valid, W1, b1, g1, beta1,
                       W2, b2, g2, beta2, W3, b3, g3, beta3)
